# baseline (speedup 1.0000x reference)
import functools

import jax
import jax.numpy as jnp
from jax import lax
from jax.experimental import pallas as pl
from jax.experimental.pallas import tpu as pltpu
from jax.experimental.pallas import tpu_sc as plsc

NUM_CORES = 2
NUM_SUBCORES = 16
NW = NUM_CORES * NUM_SUBCORES
LANES = 16


@functools.lru_cache(maxsize=None)
def _make_embed(n_rows, vocab, maxlen, embed, chunk_rows, interpret=False):
    assert n_rows % (NW * chunk_rows) == 0
    assert maxlen % chunk_rows == 0 or chunk_rows % maxlen == 0
    assert embed % LANES == 0
    rows_per_w = n_rows // NW
    n_chunks = rows_per_w // chunk_rows
    assert n_chunks % 2 == 0
    groups = embed // LANES
    mesh = plsc.VectorSubcoreMesh(
        core_axis_name="c", subcore_axis_name="s",
        num_cores=NUM_CORES, num_subcores=NUM_SUBCORES)

    @functools.partial(
        pl.kernel,
        out_type=jax.ShapeDtypeStruct((n_rows * embed,), jnp.float32),
        mesh=mesh,
        scratch_types=[
            pltpu.VMEM((rows_per_w,), jnp.int32),
            pltpu.VMEM((chunk_rows, embed), jnp.float32),
            pltpu.VMEM((chunk_rows, embed), jnp.float32),
            pltpu.VMEM((chunk_rows * embed,), jnp.float32),
            pltpu.VMEM((chunk_rows * embed,), jnp.float32),
            pltpu.VMEM((maxlen, embed), jnp.float32),
            pltpu.SemaphoreType.DMA,
            pltpu.SemaphoreType.DMA,
            pltpu.SemaphoreType.DMA,
            pltpu.SemaphoreType.DMA,
        ],
        compiler_params=pltpu.CompilerParams(use_tc_tiling_on_sc=False),
        interpret=interpret,
    )
    def embed_kernel(x_hbm, tok_hbm, pos_hbm, out_hbm, idx_v, rowsg0, rowsg1,
                     rowso0, rowso1, pos_v, sg0, sg1, so0, so1):
        wid = lax.axis_index("s") * NUM_CORES + lax.axis_index("c")
        base = wid * rows_per_w
        rowsg = (rowsg0, rowsg1)
        rowso = (rowso0, rowso1)
        sg = (sg0, sg1)
        so = (so0, so1)

        pltpu.sync_copy(pos_hbm, pos_v)
        pltpu.sync_copy(x_hbm.at[pl.ds(base, rows_per_w)], idx_v)

        def gather(g, b):
            pltpu.async_copy(
                tok_hbm.at[idx_v.at[pl.ds(g * chunk_rows, chunk_rows)]],
                rowsg[b], sg[b])

        def wait_gather(b):
            pltpu.make_async_copy(
                tok_hbm.at[idx_v.at[pl.ds(0, chunk_rows)]], rowsg[b],
                sg[b]).wait()

        def put_out(g, b):
            pltpu.async_copy(
                rowso[b],
                out_hbm.at[pl.ds((base + g * chunk_rows) * embed,
                                 chunk_rows * embed)],
                so[b])

        def wait_out(b):
            pltpu.make_async_copy(
                rowso[b], out_hbm.at[pl.ds(0, chunk_rows * embed)],
                so[b]).wait()

        gather(0, 0)

        def step(g, b):
            wait_gather(b)

            @pl.when(g + 1 < n_chunks)
            def _():
                gather(g + 1, 1 - b)

            @pl.when(g >= 2)
            def _():
                wait_out(b)

            def add_body(s, carry):
                for k in range(groups):
                    sl = pl.ds(k * LANES, LANES)
                    fsl = pl.ds(s * embed + k * LANES, LANES)
                    rowso[b][fsl] = rowsg[b][s, sl] + pos_v[s, sl]
                return carry

            lax.fori_loop(0, chunk_rows, add_body, 0, unroll=2)
            put_out(g, b)
            return b

        def pair(g0, carry):
            step(g0 * 2, 0)
            step(g0 * 2 + 1, 1)
            return carry

        lax.fori_loop(0, n_chunks // 2, pair, 0)
        wait_out(0)
        wait_out(1)

    return embed_kernel


def kernel(x, token_table, pos_table):
    batch, seq = x.shape
    vocab, embed = token_table.shape
    maxlen = pos_table.shape[0]
    n_rows = batch * seq
    chunk_rows = maxlen
    fn = _make_embed(n_rows, vocab, maxlen, embed, chunk_rows)
    xf = x.reshape(n_rows).astype(jnp.int32)
    out1 = fn(xf, token_table, pos_table)
    return out1.reshape(batch, seq, embed)


# speedup vs baseline: 1.0178x; 1.0178x over previous
import functools

import jax
import jax.numpy as jnp
from jax import lax
from jax.experimental import pallas as pl
from jax.experimental.pallas import tpu as pltpu
from jax.experimental.pallas import tpu_sc as plsc

NUM_CORES = 2
NUM_SUBCORES = 16
NW = NUM_CORES * NUM_SUBCORES
LANES = 16


@functools.lru_cache(maxsize=None)
def _make_embed(n_rows, vocab, maxlen, embed, chunk_rows, interpret=False):
    assert n_rows % (NW * chunk_rows) == 0
    assert chunk_rows % maxlen == 0 and chunk_rows % 2 == 0
    assert embed % LANES == 0
    rows_per_w = n_rows // NW
    n_chunks = rows_per_w // chunk_rows
    assert n_chunks % 2 == 0
    reps = chunk_rows // maxlen
    groups = embed // LANES
    mesh = plsc.VectorSubcoreMesh(
        core_axis_name="c", subcore_axis_name="s",
        num_cores=NUM_CORES, num_subcores=NUM_SUBCORES)

    @functools.partial(
        pl.kernel,
        out_type=jax.ShapeDtypeStruct((n_rows, embed), jnp.float32),
        mesh=mesh,
        scratch_types=[
            pltpu.VMEM((rows_per_w,), jnp.int32),
            pltpu.VMEM((chunk_rows, embed), jnp.float32),
            pltpu.VMEM((chunk_rows, embed), jnp.float32),
            pltpu.VMEM((maxlen, embed), jnp.float32),
            pltpu.SemaphoreType.DMA,
            pltpu.SemaphoreType.DMA,
            pltpu.SemaphoreType.DMA,
            pltpu.SemaphoreType.DMA,
        ],
        compiler_params=pltpu.CompilerParams(use_tc_tiling_on_sc=False),
        interpret=interpret,
    )
    def embed_kernel(x_hbm, tok_hbm, pos_hbm, out_hbm, idx_v, rows0, rows1,
                     pos_v, sg0, sg1, so0, so1):
        wid = lax.axis_index("s") * NUM_CORES + lax.axis_index("c")
        base = wid * rows_per_w
        rows = (rows0, rows1)
        sg = (sg0, sg1)
        so = (so0, so1)

        pltpu.sync_copy(pos_hbm, pos_v)
        pltpu.sync_copy(x_hbm.at[pl.ds(base, rows_per_w)], idx_v)

        def gather(g, b):
            pltpu.async_copy(
                tok_hbm.at[idx_v.at[pl.ds(g * chunk_rows, chunk_rows)]],
                rows[b], sg[b])

        def wait_gather(b):
            pltpu.make_async_copy(
                tok_hbm.at[idx_v.at[pl.ds(0, chunk_rows)]], rows[b],
                sg[b]).wait()

        out2d = out_hbm

        def put_out_half(g, b, r):
            pltpu.async_copy(
                rows[b].at[pl.ds(r * maxlen, maxlen)],
                out2d.at[pl.ds(base + g * chunk_rows + r * maxlen, maxlen)],
                so[b])

        def wait_out(b):
            for r in range(reps):
                pltpu.make_async_copy(
                    rows[b].at[pl.ds(r * maxlen, maxlen)],
                    out2d.at[pl.ds(0, maxlen)], so[b]).wait()

        gather(0, 0)

        def step(g, b):
            wait_gather(b)

            @pl.when(g + 1 < n_chunks)
            def _():
                @pl.when(g >= 1)
                def _():
                    wait_out(1 - b)

                gather(g + 1, 1 - b)

            for r in range(reps):
                def add_body(s, carry, r=r):
                    j = s + r * maxlen
                    for k in range(groups):
                        sl = pl.ds(k * LANES, LANES)
                        rows[b][j, sl] = rows[b][j, sl] + pos_v[s, sl]
                    return carry

                lax.fori_loop(0, maxlen, add_body, 0, unroll=4)
                put_out_half(g, b, r)
            return b

        def pair(g0, carry):
            step(g0 * 2, 0)
            step(g0 * 2 + 1, 1)
            return carry

        lax.fori_loop(0, n_chunks // 2, pair, 0)
        wait_out(0)
        wait_out(1)

    return embed_kernel


def kernel(x, token_table, pos_table):
    batch, seq = x.shape
    vocab, embed = token_table.shape
    maxlen = pos_table.shape[0]
    n_rows = batch * seq
    chunk_rows = 2 * maxlen
    fn = _make_embed(n_rows, vocab, maxlen, embed, chunk_rows)
    xf = x.reshape(n_rows).astype(jnp.int32)
    out2 = fn(xf, token_table, pos_table)
    return out2.reshape(batch, seq, embed)


# speedup vs baseline: 1.1461x; 1.1260x over previous
import functools

import jax
import jax.numpy as jnp
from jax import lax
from jax.experimental import pallas as pl
from jax.experimental.pallas import tpu as pltpu
from jax.experimental.pallas import tpu_sc as plsc

NUM_CORES = 2
NUM_SUBCORES = 16
NW = NUM_CORES * NUM_SUBCORES
LANES = 16


@functools.lru_cache(maxsize=None)
def _make_embed(n_rows, vocab, maxlen, embed, chunk_rows, interpret=False):
    assert n_rows % (NW * chunk_rows) == 0
    assert chunk_rows % maxlen == 0 and chunk_rows % 2 == 0
    assert embed % LANES == 0
    rows_per_w = n_rows // NW
    n_chunks = rows_per_w // chunk_rows
    assert n_chunks % 2 == 0
    reps = chunk_rows // maxlen
    groups = embed // LANES
    mesh = plsc.VectorSubcoreMesh(
        core_axis_name="c", subcore_axis_name="s",
        num_cores=NUM_CORES, num_subcores=NUM_SUBCORES)

    @functools.partial(
        pl.kernel,
        out_type=jax.ShapeDtypeStruct((n_rows, embed), jnp.float32),
        mesh=mesh,
        scratch_types=[
            pltpu.VMEM((rows_per_w,), jnp.int32),
            pltpu.VMEM((chunk_rows, embed), jnp.float32),
            pltpu.VMEM((chunk_rows, embed), jnp.float32),
            pltpu.VMEM((maxlen, embed), jnp.float32),
            pltpu.SemaphoreType.DMA,
            pltpu.SemaphoreType.DMA,
            pltpu.SemaphoreType.DMA,
            pltpu.SemaphoreType.DMA,
        ],
        compiler_params=pltpu.CompilerParams(use_tc_tiling_on_sc=False),
        interpret=interpret,
    )
    def embed_kernel(x_hbm, tok_hbm, pos_hbm, out_hbm, idx_v, rows0, rows1,
                     pos_v, sg0, sg1, so0, so1):
        wid = lax.axis_index("s") * NUM_CORES + lax.axis_index("c")
        base = wid * rows_per_w
        rows = (rows0, rows1)
        sg = (sg0, sg1)
        so = (so0, so1)

        pltpu.sync_copy(pos_hbm, pos_v)
        pltpu.sync_copy(x_hbm.at[pl.ds(base, rows_per_w)], idx_v)

        def gather(g, b):
            pltpu.async_copy(
                tok_hbm.at[idx_v.at[pl.ds(g * chunk_rows, chunk_rows)]],
                rows[b], sg[b])

        def wait_gather(b):
            pltpu.make_async_copy(
                tok_hbm.at[idx_v.at[pl.ds(0, chunk_rows)]], rows[b],
                sg[b]).wait()

        out2d = out_hbm

        def put_out(g, b):
            pltpu.async_copy(
                rows[b],
                out2d.at[pl.ds(base + g * chunk_rows, chunk_rows)],
                so[b])

        def wait_out(b):
            pltpu.make_async_copy(
                rows[b], out2d.at[pl.ds(0, chunk_rows)], so[b]).wait()

        gather(0, 0)

        def step(g, b):
            wait_gather(b)

            @pl.when(g + 1 < n_chunks)
            def _():
                @pl.when(g >= 1)
                def _():
                    wait_out(1 - b)

                gather(g + 1, 1 - b)

            def add_body(s, carry):
                for k in range(groups):
                    sl = pl.ds(k * LANES, LANES)
                    p = pos_v[s, sl]
                    for r in range(reps):
                        j = s + r * maxlen
                        rows[b][j, sl] = rows[b][j, sl] + p
                return carry

            lax.fori_loop(0, maxlen, add_body, 0, unroll=2)
            put_out(g, b)
            return b

        def pair(g0, carry):
            step(g0 * 2, 0)
            step(g0 * 2 + 1, 1)
            return carry

        lax.fori_loop(0, n_chunks // 2, pair, 0)
        wait_out(0)
        wait_out(1)

    return embed_kernel


def kernel(x, token_table, pos_table):
    batch, seq = x.shape
    vocab, embed = token_table.shape
    maxlen = pos_table.shape[0]
    n_rows = batch * seq
    chunk_rows = 2 * maxlen
    fn = _make_embed(n_rows, vocab, maxlen, embed, chunk_rows)
    xf = x.reshape(n_rows).astype(jnp.int32)
    out2 = fn(xf, token_table, pos_table)
    return out2.reshape(batch, seq, embed)
